# baseline (speedup 1.0000x reference)
import functools

import jax
import jax.numpy as jnp
from jax import lax
from jax.experimental import pallas as pl
from jax.experimental.pallas import tpu as pltpu
from jax.experimental.pallas import tpu_sc as plsc

N = 50000
E = 800000
B = 16384
D = 64

NPAD = 50176
EPAD = 802816
NWORK = 32
EPW = EPAD // NWORK
NBLK = EPW // 128
NBLKT = NWORK * NBLK
STRIPE = NPAD // 16
NRANGE = 4
NR = NPAD // NRANGE
RSTRIPE = NR // 16


def _lrelu(x):
    return jnp.maximum(x, 0.2 * x)


def _tc_pre1_body(tbl, w1, ast, adt, xpr, asr, adr, astar):
    xp = jnp.dot(tbl[...], w1[...], preferred_element_type=jnp.float32)
    xpr[...] = xp
    a_s = lax.dot_general(ast[...], xp, (((1,), (1,)), ((), ())),
                          preferred_element_type=jnp.float32)
    a_d = lax.dot_general(adt[...], xp, (((1,), (1,)), ((), ())),
                          preferred_element_type=jnp.float32)
    asr[...] = a_s
    adr[...] = a_d
    bm = jnp.max(a_s, axis=1)
    prev = jnp.where(pl.program_id(0) == 0,
                     jnp.full((4, 128), -3.4e38, jnp.float32), astar[...])
    astar[...] = jnp.maximum(prev, bm[:, None])


def _tc_pre1(table_p, W1, As1T, Ad1T):
    grid = NPAD // 256
    return pl.pallas_call(
        _tc_pre1_body,
        grid=(grid,),
        in_specs=[
            pl.BlockSpec((256, D), lambda i: (i, 0)),
            pl.BlockSpec((D, 128), lambda i: (0, 0)),
            pl.BlockSpec((4, 128), lambda i: (0, 0)),
            pl.BlockSpec((4, 128), lambda i: (0, 0)),
        ],
        out_specs=[
            pl.BlockSpec((256, 128), lambda i: (i, 0)),
            pl.BlockSpec((4, 256), lambda i: (0, i)),
            pl.BlockSpec((4, 256), lambda i: (0, i)),
            pl.BlockSpec((4, 128), lambda i: (0, 0)),
        ],
        out_shape=[
            jax.ShapeDtypeStruct((NPAD, 128), jnp.float32),
            jax.ShapeDtypeStruct((4, NPAD), jnp.float32),
            jax.ShapeDtypeStruct((4, NPAD), jnp.float32),
            jax.ShapeDtypeStruct((4, 128), jnp.float32),
        ],
    )(table_p, W1, As1T, Ad1T)


def _tc_comb1_body(accs, dens, xpr, asr, adr, astar, b1r, w2, as2t, ad2t,
                   xpr2, asr2, adr2, astar2):
    num = accs[0] + accs[1]
    den = dens[0] + dens[1]
    a_s = asr[...]
    a_d = adr[...]
    m = _lrelu(astar[:, 0:1] + a_d)
    ws = jnp.exp(_lrelu(a_s + a_d) - m)
    xp = xpr[...]
    cols = []
    for h in range(4):
        sl = slice(h * 32, (h + 1) * 32)
        numh = num[:, sl] + ws[h][:, None] * xp[:, sl]
        cols.append(numh / (den[h] + ws[h] + 1e-16)[:, None])
    h1 = jnp.concatenate(cols, axis=1) + b1r[...]
    h1 = jnp.where(h1 > 0, h1, jnp.exp(jnp.minimum(h1, 0.0)) - 1.0)
    xp2 = jnp.dot(h1, w2[...], preferred_element_type=jnp.float32)
    xpr2[:, 0:64] = xp2
    xpr2[:, 64:128] = jnp.zeros((256, 64), jnp.float32)
    a_s2 = lax.dot_general(as2t[...], xp2, (((1,), (1,)), ((), ())),
                           preferred_element_type=jnp.float32)
    a_d2 = lax.dot_general(ad2t[...], xp2, (((1,), (1,)), ((), ())),
                           preferred_element_type=jnp.float32)
    asr2[...] = a_s2
    adr2[...] = a_d2
    bm = jnp.max(a_s2, axis=1)
    prev = jnp.where(pl.program_id(0) == 0,
                     jnp.full((1, 128), -3.4e38, jnp.float32), astar2[...])
    astar2[...] = jnp.maximum(prev, bm[:, None])


def _tc_comb1(accs, dens, xpr, asr, adr, astar, b1r, W2, As2T, Ad2T):
    grid = NPAD // 256
    return pl.pallas_call(
        _tc_comb1_body,
        grid=(grid,),
        in_specs=[
            pl.BlockSpec((2, 256, 128), lambda i: (0, i, 0)),
            pl.BlockSpec((2, 4, 256), lambda i: (0, 0, i)),
            pl.BlockSpec((256, 128), lambda i: (i, 0)),
            pl.BlockSpec((4, 256), lambda i: (0, i)),
            pl.BlockSpec((4, 256), lambda i: (0, i)),
            pl.BlockSpec((4, 128), lambda i: (0, 0)),
            pl.BlockSpec((1, 128), lambda i: (0, 0)),
            pl.BlockSpec((128, 64), lambda i: (0, 0)),
            pl.BlockSpec((1, 64), lambda i: (0, 0)),
            pl.BlockSpec((1, 64), lambda i: (0, 0)),
        ],
        out_specs=[
            pl.BlockSpec((256, 128), lambda i: (i, 0)),
            pl.BlockSpec((1, 256), lambda i: (0, i)),
            pl.BlockSpec((1, 256), lambda i: (0, i)),
            pl.BlockSpec((1, 128), lambda i: (0, 0)),
        ],
        out_shape=[
            jax.ShapeDtypeStruct((NPAD, 128), jnp.float32),
            jax.ShapeDtypeStruct((1, NPAD), jnp.float32),
            jax.ShapeDtypeStruct((1, NPAD), jnp.float32),
            jax.ShapeDtypeStruct((1, 128), jnp.float32),
        ],
    )(accs, dens, xpr, asr, adr, astar, b1r, W2, As2T, Ad2T)


def _tc_comb2_body(accs, dens, xpr, asr, adr, astar, b2r, h2):
    num = accs[0] + accs[1]
    den = dens[0] + dens[1]
    a_s = asr[...]
    a_d = adr[...]
    m = _lrelu(astar[:, 0:1] + a_d)
    ws = jnp.exp(_lrelu(a_s + a_d) - m)
    xp = xpr[...]
    out = (num[:, 0:64] + ws[0][:, None] * xp[:, 0:64])
    out = out / (den[0] + ws[0] + 1e-16)[:, None] + b2r[...]
    h2[:, 0:64] = out
    h2[:, 64:128] = jnp.zeros((256, 64), jnp.float32)


def _tc_comb2(accs, dens, xpr, asr, adr, astar, b2r):
    grid = NPAD // 256
    return pl.pallas_call(
        _tc_comb2_body,
        grid=(grid,),
        in_specs=[
            pl.BlockSpec((2, 256, 128), lambda i: (0, i, 0)),
            pl.BlockSpec((2, 1, 256), lambda i: (0, 0, i)),
            pl.BlockSpec((256, 128), lambda i: (i, 0)),
            pl.BlockSpec((1, 256), lambda i: (0, i)),
            pl.BlockSpec((1, 256), lambda i: (0, i)),
            pl.BlockSpec((1, 128), lambda i: (0, 0)),
            pl.BlockSpec((1, 64), lambda i: (0, 0)),
        ],
        out_specs=pl.BlockSpec((256, 128), lambda i: (i, 0)),
        out_shape=jax.ShapeDtypeStruct((NPAD, 128), jnp.float32),
    )(accs, dens, xpr, asr, adr, astar, b2r)


def _tc_mlp_body(u, v, p1u, p1v, pb1, pw2, pb2, out):
    z = jnp.dot(u[...], p1u[...], preferred_element_type=jnp.float32)
    z = z + jnp.dot(v[...], p1v[...], preferred_element_type=jnp.float32)
    z = jnp.maximum(z + pb1[...], 0.0)
    o = jnp.dot(z, pw2[...], preferred_element_type=jnp.float32) + pb2[...]
    out[...] = jax.nn.sigmoid(o)


def _tc_mlp(u, v, P1u, P1v, Pb1r, Pw2p, Pb2r):
    grid = B // 512
    return pl.pallas_call(
        _tc_mlp_body,
        grid=(grid,),
        in_specs=[
            pl.BlockSpec((512, 128), lambda i: (i, 0)),
            pl.BlockSpec((512, 128), lambda i: (i, 0)),
            pl.BlockSpec((128, 128), lambda i: (0, 0)),
            pl.BlockSpec((128, 128), lambda i: (0, 0)),
            pl.BlockSpec((1, 128), lambda i: (0, 0)),
            pl.BlockSpec((128, 8), lambda i: (0, 0)),
            pl.BlockSpec((1, 8), lambda i: (0, 0)),
        ],
        out_specs=pl.BlockSpec((512, 8), lambda i: (i, 0)),
        out_shape=jax.ShapeDtypeStruct((B, 8), jnp.float32),
    )(u, v, P1u, P1v, Pb1r, Pw2p, Pb2r)


def _sc_mesh():
    return plsc.VectorSubcoreMesh(core_axis_name="c", subcore_axis_name="s")


def _sc_weights(edge_f, asT, adT, astar, heads):
    @functools.partial(
        pl.kernel,
        out_type=[
            jax.ShapeDtypeStruct((NBLKT * heads * 128,), jnp.float32),
            jax.ShapeDtypeStruct((2 * heads * NPAD,), jnp.float32),
        ],
        mesh=_sc_mesh(),
        compiler_params=pltpu.CompilerParams(needs_layout_passes=False),
        scratch_types=[
            pltpu.VMEM((NPAD,), jnp.float32),
            pltpu.VMEM((NPAD,), jnp.float32),
            pltpu.VMEM((16,), jnp.float32),
            pltpu.VMEM((2, 256), jnp.int32),
            pltpu.VMEM((2, 128), jnp.int32),
            pltpu.VMEM((2, 128), jnp.float32),
            pltpu.VMEM((3136,), jnp.float32),
            pltpu.VMEM((3136,), jnp.float32),
            pltpu.SemaphoreType.DMA((2,)),
            pltpu.SemaphoreType.DMA((2,)),
            pltpu.SemaphoreType.DMA((2,)),
            pltpu.VMEM_SHARED((heads * NPAD,), jnp.float32),
        ],
    )
    def k(edge_hbm, asT_hbm, adT_hbm, astar_hbm, w_hbm, den_hbm,
          as_t, ad_t, abuf, sdbuf, ibuf, wbuf, zflat, bflat,
          esem, wwsem, dssem, den_sp):
        cid = lax.axis_index("c")
        sid = lax.axis_index("s")
        ebase = (cid * 16 + sid) * EPW

        def _z1(i, _):
            zflat[pl.ds(i * 16, 16)] = jnp.zeros((16,), jnp.float32)
            return 0
        lax.fori_loop(0, 196, _z1, 0)

        def _head(h, _):
            pltpu.sync_copy(asT_hbm.at[pl.ds(h * NPAD, NPAD)], as_t)
            pltpu.sync_copy(adT_hbm.at[pl.ds(h * NPAD, NPAD)], ad_t)
            pltpu.sync_copy(astar_hbm.at[pl.ds(h * 128, 16)], abuf)
            pltpu.sync_copy(zflat, den_sp.at[pl.ds(h * NPAD + sid * STRIPE,
                                                   STRIPE)])
            plsc.subcore_barrier()

            def _start(b, p):
                blkid = (cid * 16 + sid) * NBLK + b
                pltpu.async_copy(edge_hbm.at[pl.ds(blkid * 256, 256)],
                                 sdbuf.at[p], esem.at[p])

            _start(0, 0)

            def _blk(b, _):
                p = lax.rem(b, 2)
                blkid = (cid * 16 + sid) * NBLK + b

                _start(jnp.minimum(b + 1, NBLK - 1), 1 - p)
                pltpu.make_async_copy(edge_hbm.at[pl.ds(0, 256)],
                                      sdbuf.at[p], esem.at[p]).wait()

                def _wv(i, _):
                    pltpu.make_async_copy(
                        wbuf.at[p], w_hbm.at[pl.ds(0, 128)],
                        wwsem.at[p]).wait()
                    pltpu.make_async_copy(
                        wbuf.at[p], den_sp.at[pl.ds(0, 128)],
                        dssem.at[p]).wait()
                    return 0
                lax.fori_loop(0, jnp.where(b >= 2, 1, 0), _wv, 0)

                av16 = abuf[...]
                for g in range(8):
                    sv = sdbuf[p, pl.ds(g * 16, 16)]
                    dv = sdbuf[p, pl.ds(128 + g * 16, 16)]
                    a1 = plsc.load_gather(as_t, [sv])
                    a2 = plsc.load_gather(ad_t, [dv])
                    alpha = _lrelu(a1 + a2)
                    mm = _lrelu(av16 + a2)
                    wbuf[p, pl.ds(g * 16, 16)] = jnp.exp(alpha - mm)
                    ibuf[p, pl.ds(g * 16, 16)] = dv + h * NPAD
                pltpu.async_copy(
                    wbuf.at[p],
                    w_hbm.at[pl.ds((blkid * heads + h) * 128, 128)],
                    wwsem.at[p])
                pltpu.async_copy(wbuf.at[p], den_sp.at[ibuf.at[p]],
                                 dssem.at[p], add=True)
                return 0
            lax.fori_loop(0, NBLK, _blk, 0)
            pltpu.make_async_copy(edge_hbm.at[pl.ds(0, 256)],
                                  sdbuf.at[0], esem.at[0]).wait()
            for p in range(2):
                pltpu.make_async_copy(wbuf.at[p], w_hbm.at[pl.ds(0, 128)],
                                      wwsem.at[p]).wait()
                pltpu.make_async_copy(wbuf.at[p], den_sp.at[pl.ds(0, 128)],
                                      dssem.at[p]).wait()
            plsc.subcore_barrier()
            pltpu.sync_copy(
                den_sp.at[pl.ds(h * NPAD + sid * STRIPE, STRIPE)], bflat)
            pltpu.sync_copy(
                bflat,
                den_hbm.at[pl.ds(cid * heads * NPAD + h * NPAD + sid * STRIPE,
                                 STRIPE)])
            return 0
        lax.fori_loop(0, heads, _head, 0)

    return k(edge_f, asT, adT, astar)


def _sc_aggregate(edge_f, w_f, xp, heads):
    @functools.partial(
        pl.kernel,
        out_type=jax.ShapeDtypeStruct((2, NRANGE, NR, 128), jnp.float32),
        mesh=_sc_mesh(),
        compiler_params=pltpu.CompilerParams(needs_layout_passes=False),
        scratch_types=[
            pltpu.VMEM((2, 256), jnp.int32),
            pltpu.VMEM((2, 4 * 128), jnp.float32),
            pltpu.VMEM((256,), jnp.int32),
            pltpu.VMEM((256,), jnp.int32),
            pltpu.VMEM((4 * 256,), jnp.float32),
            pltpu.VMEM((128,), jnp.int32),
            pltpu.VMEM((128,), jnp.int32),
            pltpu.VMEM((4 * 128,), jnp.float32),
            pltpu.VMEM((128, 128), jnp.float32),
            pltpu.SemaphoreType.DMA((2,)),
            pltpu.SemaphoreType.DMA((2,)),
            pltpu.SemaphoreType.DMA,
            pltpu.VMEM_SHARED((NR, 128), jnp.float32),
        ],
    )
    def k(edge_hbm, w_hbm, xp_hbm, acc_hbm,
          sdbuf, wstg, csrc, cdst, cw, fsrc, fdst, fcw, rows,
          esem, wsem, gsem, acc_sp):
        cid = lax.axis_index("c")
        sid = lax.axis_index("s")
        nh = heads

        for g in range(16):
            csrc[pl.ds(g * 16, 16)] = jnp.zeros((16,), jnp.int32)
            cdst[pl.ds(g * 16, 16)] = jnp.zeros((16,), jnp.int32)
        for g in range(16 * nh):
            cw[pl.ds(g * 16, 16)] = jnp.zeros((16,), jnp.float32)

        def _fcomplete():
            pltpu.make_async_copy(xp_hbm.at[fsrc], rows, gsem).wait()

            def _scale(g, _):
                wvecs = [fcw[pl.ds(q * 128 + g * 16, 16)] for q in range(nh)]
                for e in range(16):
                    i = g * 16 + e
                    for q in range(nh):
                        wv = jnp.full((16,), wvecs[q][e], jnp.float32)
                        rng = (range(2 * q, 2 * q + 2) if nh == 4
                               else range(8))
                        for j in rng:
                            rows[i, pl.ds(j * 16, 16)] = (
                                rows[i, pl.ds(j * 16, 16)] * wv)
                return 0
            lax.fori_loop(0, 8, _scale, 0)
            pltpu.sync_copy(rows, acc_sp.at[fdst], add=True)

        def _fissue():
            for j in range(8):
                fsrc[pl.ds(j * 16, 16)] = csrc[pl.ds(j * 16, 16)]
                fdst[pl.ds(j * 16, 16)] = cdst[pl.ds(j * 16, 16)]
                csrc[pl.ds(j * 16, 16)] = csrc[pl.ds(128 + j * 16, 16)]
                cdst[pl.ds(j * 16, 16)] = cdst[pl.ds(128 + j * 16, 16)]
            for q in range(nh):
                for j in range(8):
                    fcw[pl.ds(q * 128 + j * 16, 16)] = (
                        cw[pl.ds(q * 256 + j * 16, 16)])
                    cw[pl.ds(q * 256 + j * 16, 16)] = (
                        cw[pl.ds(q * 256 + 128 + j * 16, 16)])
            pltpu.async_copy(xp_hbm.at[fsrc], rows, gsem)

        def _range(r, _):
            lo = r * NR

            def _z0(i, _):
                for j in range(8):
                    rows[i, pl.ds(j * 16, 16)] = jnp.zeros((16,), jnp.float32)
                return 0
            lax.fori_loop(0, 112, _z0, 0)

            def _zacc(i, _):
                pltpu.sync_copy(rows.at[pl.ds(0, 112)],
                                acc_sp.at[pl.ds(sid * RSTRIPE + i * 112, 112)])
                return 0
            lax.fori_loop(0, 7, _zacc, 0)
            plsc.subcore_barrier()

            for j in range(8):
                fsrc[pl.ds(j * 16, 16)] = jnp.zeros((16,), jnp.int32)
                fdst[pl.ds(j * 16, 16)] = jnp.zeros((16,), jnp.int32)
            for j in range(8 * nh):
                fcw[pl.ds(j * 16, 16)] = jnp.zeros((16,), jnp.float32)
            pltpu.async_copy(xp_hbm.at[fsrc], rows, gsem)

            def _start(b, p):
                blkid = (cid * 16 + sid) * NBLK + b
                pltpu.async_copy(edge_hbm.at[pl.ds(blkid * 256, 256)],
                                 sdbuf.at[p], esem.at[p])
                pltpu.async_copy(
                    w_hbm.at[pl.ds(blkid * nh * 128, nh * 128)],
                    wstg.at[p, pl.ds(0, nh * 128)], wsem.at[p])

            _start(0, 0)

            def _blk(b, off):
                p = lax.rem(b, 2)
                _start(jnp.minimum(b + 1, NBLK - 1), 1 - p)
                pltpu.make_async_copy(edge_hbm.at[pl.ds(0, 256)],
                                      sdbuf.at[p], esem.at[p]).wait()
                pltpu.make_async_copy(
                    w_hbm.at[pl.ds(0, nh * 128)],
                    wstg.at[p, pl.ds(0, nh * 128)], wsem.at[p]).wait()
                for g in range(8):
                    sv = sdbuf[p, pl.ds(g * 16, 16)]
                    dv = sdbuf[p, pl.ds(128 + g * 16, 16)]
                    msk = (dv >= lo) & (dv < lo + NR)
                    plsc.store_compressed(csrc.at[pl.ds(off, 16)], sv,
                                          mask=msk)
                    plsc.store_compressed(cdst.at[pl.ds(off, 16)], dv - lo,
                                          mask=msk)
                    for q in range(nh):
                        wv = wstg[p, pl.ds(q * 128 + g * 16, 16)]
                        plsc.store_compressed(
                            cw.at[pl.ds(q * 256 + off, 16)], wv, mask=msk)
                    cnt = plsc.all_reduce_population_count(msk)
                    off = off + cnt[0]

                def _doflush(i, _):
                    _fcomplete()
                    _fissue()
                    return 0
                nfl = jnp.where(off >= 128, 1, 0)
                lax.fori_loop(0, nfl, _doflush, 0)
                off = jnp.where(off >= 128, off - 128, off)
                return off
            off = lax.fori_loop(0, NBLK, _blk, jnp.int32(0))
            pltpu.make_async_copy(edge_hbm.at[pl.ds(0, 256)],
                                  sdbuf.at[0], esem.at[0]).wait()
            pltpu.make_async_copy(w_hbm.at[pl.ds(0, nh * 128)],
                                  wstg.at[0, pl.ds(0, nh * 128)],
                                  wsem.at[0]).wait()

            iota = lax.iota(jnp.int32, 16)
            for g in range(8):
                keep = (iota + g * 16) < off
                for q in range(nh):
                    wv = cw[pl.ds(q * 256 + g * 16, 16)]
                    cw[pl.ds(q * 256 + g * 16, 16)] = jnp.where(
                        keep, wv, jnp.zeros((16,), jnp.float32))
            _fcomplete()
            _fissue()
            _fcomplete()
            plsc.subcore_barrier()

            def _dump(i, _):
                o = sid * RSTRIPE + i * 112
                pltpu.sync_copy(acc_sp.at[pl.ds(o, 112)],
                                rows.at[pl.ds(0, 112)])
                pltpu.sync_copy(rows.at[pl.ds(0, 112)],
                                acc_hbm.at[cid, r, pl.ds(o, 112)])
                return 0
            lax.fori_loop(0, 7, _dump, 0)
            return 0
        lax.fori_loop(0, NRANGE, _range, 0)

    return k(edge_f, w_f, xp)


def _sc_gather(h2, uid, vid):
    @functools.partial(
        pl.kernel,
        out_type=[
            jax.ShapeDtypeStruct((B, 128), jnp.float32),
            jax.ShapeDtypeStruct((B, 128), jnp.float32),
        ],
        mesh=_sc_mesh(),
        compiler_params=pltpu.CompilerParams(needs_layout_passes=False),
        scratch_types=[
            pltpu.VMEM((128,), jnp.int32),
            pltpu.VMEM((128, 128), jnp.float32),
        ],
    )
    def k(h2_hbm, uid_hbm, vid_hbm, u_hbm, v_hbm, ibuf, rbuf):
        cid = lax.axis_index("c")
        sid = lax.axis_index("s")
        base = (cid * 16 + sid) * (B // NWORK)

        def _blk(b, _):
            off = base + b * 128
            pltpu.sync_copy(uid_hbm.at[pl.ds(off, 128)], ibuf)
            pltpu.sync_copy(h2_hbm.at[ibuf], rbuf)
            pltpu.sync_copy(rbuf, u_hbm.at[pl.ds(off, 128)])
            pltpu.sync_copy(vid_hbm.at[pl.ds(off, 128)], ibuf)
            pltpu.sync_copy(h2_hbm.at[ibuf], rbuf)
            pltpu.sync_copy(rbuf, v_hbm.at[pl.ds(off, 128)])
            return 0
        lax.fori_loop(0, (B // NWORK) // 128, _blk, 0)

    return k(h2, uid, vid)


def kernel(x, edge_index, user_indices, item_indices, table, W1, a_src1,
           a_dst1, b1, W2, a_src2, a_dst2, b2, Pw1, Pb1, Pw2, Pb2):
    f32 = jnp.float32
    table_p = jnp.pad(table, ((0, NPAD - N), (0, 0)))
    npad_ids = jnp.arange(EPAD - E, dtype=jnp.int32)
    pad_src = npad_ids % N
    pad_dst = N + (npad_ids % (NPAD - N))
    edge_p = jnp.concatenate(
        [edge_index, jnp.stack([pad_src, pad_dst])], axis=1)
    edge_f = edge_p.reshape(2, NBLKT, 128).transpose(1, 0, 2).reshape(-1)

    eye4 = jnp.eye(4, dtype=f32)
    As1T = (eye4[:, :, None] * a_src1[None]).reshape(4, 128)
    Ad1T = (eye4[:, :, None] * a_dst1[None]).reshape(4, 128)
    b1r = b1.reshape(1, 128)
    b2r = b2.reshape(1, 64)
    P1u = jnp.pad(Pw1[:64], ((0, 64), (0, 0)))
    P1v = jnp.pad(Pw1[64:], ((0, 64), (0, 0)))
    Pb1r = Pb1.reshape(1, 128)
    Pw2p = jnp.pad(Pw2, ((0, 0), (0, 7)))
    Pb2r = jnp.pad(Pb2.reshape(1, 1), ((0, 0), (0, 7)))

    xp1, asT1, adT1, astar1 = _tc_pre1(table_p, W1, As1T, Ad1T)
    w1f, den1 = _sc_weights(edge_f, asT1.reshape(-1), adT1.reshape(-1),
                            astar1.reshape(-1), 4)
    acc1 = _sc_aggregate(edge_f, w1f, xp1, 4)
    xp2, asT2, adT2, astar2 = _tc_comb1(
        acc1.reshape(2, NPAD, 128), den1.reshape(2, 4, NPAD), xp1,
        asT1, adT1, astar1, b1r, W2, a_src2, a_dst2)
    w2f, den2 = _sc_weights(edge_f, asT2.reshape(-1), adT2.reshape(-1),
                            astar2.reshape(-1), 1)
    acc2 = _sc_aggregate(edge_f, w2f, xp2, 1)
    h2 = _tc_comb2(acc2.reshape(2, NPAD, 128), den2.reshape(2, 1, NPAD),
                   xp2, asT2, adT2, astar2, b2r)
    u, v = _sc_gather(h2, user_indices, item_indices)
    out = _tc_mlp(u, v, P1u, P1v, Pb1r, Pw2p, Pb2r)
    return out[:, 0]


# speedup vs baseline: 1.2593x; 1.2593x over previous
import functools

import jax
import jax.numpy as jnp
from jax import lax
from jax.experimental import pallas as pl
from jax.experimental.pallas import tpu as pltpu
from jax.experimental.pallas import tpu_sc as plsc

N = 50000
E = 800000
B = 16384
D = 64

NPAD = 50176
EPAD = 802816
NWORK = 32
EPW = EPAD // NWORK
NBLK = EPW // 128
NBLKT = NWORK * NBLK
STRIPE = NPAD // 16
NRANGE = 4
NR = NPAD // NRANGE
RSTRIPE = NR // 16


def _lrelu(x):
    return jnp.maximum(x, 0.2 * x)


def _tc_pre1_body(tbl, w1, ast, adt, xpr, asr, adr, astar):
    xp = jnp.dot(tbl[...], w1[...], preferred_element_type=jnp.float32)
    xpr[...] = xp
    a_s = lax.dot_general(ast[...], xp, (((1,), (1,)), ((), ())),
                          preferred_element_type=jnp.float32)
    a_d = lax.dot_general(adt[...], xp, (((1,), (1,)), ((), ())),
                          preferred_element_type=jnp.float32)
    asr[...] = a_s
    adr[...] = a_d
    bm = jnp.max(a_s, axis=1)
    prev = jnp.where(pl.program_id(0) == 0,
                     jnp.full((4, 128), -3.4e38, jnp.float32), astar[...])
    astar[...] = jnp.maximum(prev, bm[:, None])


def _tc_pre1(table_p, W1, As1T, Ad1T):
    grid = NPAD // 256
    return pl.pallas_call(
        _tc_pre1_body,
        grid=(grid,),
        in_specs=[
            pl.BlockSpec((256, D), lambda i: (i, 0)),
            pl.BlockSpec((D, 128), lambda i: (0, 0)),
            pl.BlockSpec((4, 128), lambda i: (0, 0)),
            pl.BlockSpec((4, 128), lambda i: (0, 0)),
        ],
        out_specs=[
            pl.BlockSpec((256, 128), lambda i: (i, 0)),
            pl.BlockSpec((4, 256), lambda i: (0, i)),
            pl.BlockSpec((4, 256), lambda i: (0, i)),
            pl.BlockSpec((4, 128), lambda i: (0, 0)),
        ],
        out_shape=[
            jax.ShapeDtypeStruct((NPAD, 128), jnp.float32),
            jax.ShapeDtypeStruct((4, NPAD), jnp.float32),
            jax.ShapeDtypeStruct((4, NPAD), jnp.float32),
            jax.ShapeDtypeStruct((4, 128), jnp.float32),
        ],
    )(table_p, W1, As1T, Ad1T)


def _tc_comb1_body(accs, dens, xpr, asr, adr, astar, b1r, w2, as2t, ad2t,
                   xpr2, asr2, adr2, astar2):
    num = accs[0] + accs[1]
    den = dens[0] + dens[1]
    a_s = asr[...]
    a_d = adr[...]
    m = _lrelu(astar[:, 0:1] + a_d)
    ws = jnp.exp(_lrelu(a_s + a_d) - m)
    xp = xpr[...]
    cols = []
    for h in range(4):
        sl = slice(h * 32, (h + 1) * 32)
        numh = num[:, sl] + ws[h][:, None] * xp[:, sl]
        cols.append(numh / (den[h] + ws[h] + 1e-16)[:, None])
    h1 = jnp.concatenate(cols, axis=1) + b1r[...]
    h1 = jnp.where(h1 > 0, h1, jnp.exp(jnp.minimum(h1, 0.0)) - 1.0)
    xp2 = jnp.dot(h1, w2[...], preferred_element_type=jnp.float32)
    xpr2[:, 0:64] = xp2
    xpr2[:, 64:128] = jnp.zeros((256, 64), jnp.float32)
    a_s2 = lax.dot_general(as2t[...], xp2, (((1,), (1,)), ((), ())),
                           preferred_element_type=jnp.float32)
    a_d2 = lax.dot_general(ad2t[...], xp2, (((1,), (1,)), ((), ())),
                           preferred_element_type=jnp.float32)
    asr2[...] = a_s2
    adr2[...] = a_d2
    bm = jnp.max(a_s2, axis=1)
    prev = jnp.where(pl.program_id(0) == 0,
                     jnp.full((1, 128), -3.4e38, jnp.float32), astar2[...])
    astar2[...] = jnp.maximum(prev, bm[:, None])


def _tc_comb1(accs, dens, xpr, asr, adr, astar, b1r, W2, As2T, Ad2T):
    grid = NPAD // 256
    return pl.pallas_call(
        _tc_comb1_body,
        grid=(grid,),
        in_specs=[
            pl.BlockSpec((2, 256, 128), lambda i: (0, i, 0)),
            pl.BlockSpec((2, 4, 256), lambda i: (0, 0, i)),
            pl.BlockSpec((256, 128), lambda i: (i, 0)),
            pl.BlockSpec((4, 256), lambda i: (0, i)),
            pl.BlockSpec((4, 256), lambda i: (0, i)),
            pl.BlockSpec((4, 128), lambda i: (0, 0)),
            pl.BlockSpec((1, 128), lambda i: (0, 0)),
            pl.BlockSpec((128, 64), lambda i: (0, 0)),
            pl.BlockSpec((1, 64), lambda i: (0, 0)),
            pl.BlockSpec((1, 64), lambda i: (0, 0)),
        ],
        out_specs=[
            pl.BlockSpec((256, 128), lambda i: (i, 0)),
            pl.BlockSpec((1, 256), lambda i: (0, i)),
            pl.BlockSpec((1, 256), lambda i: (0, i)),
            pl.BlockSpec((1, 128), lambda i: (0, 0)),
        ],
        out_shape=[
            jax.ShapeDtypeStruct((NPAD, 128), jnp.float32),
            jax.ShapeDtypeStruct((1, NPAD), jnp.float32),
            jax.ShapeDtypeStruct((1, NPAD), jnp.float32),
            jax.ShapeDtypeStruct((1, 128), jnp.float32),
        ],
    )(accs, dens, xpr, asr, adr, astar, b1r, W2, As2T, Ad2T)


def _tc_comb2_body(accs, dens, xpr, asr, adr, astar, b2r, h2):
    num = accs[0] + accs[1]
    den = dens[0] + dens[1]
    a_s = asr[...]
    a_d = adr[...]
    m = _lrelu(astar[:, 0:1] + a_d)
    ws = jnp.exp(_lrelu(a_s + a_d) - m)
    xp = xpr[...]
    out = (num[:, 0:64] + ws[0][:, None] * xp[:, 0:64])
    out = out / (den[0] + ws[0] + 1e-16)[:, None] + b2r[...]
    h2[:, 0:64] = out
    h2[:, 64:128] = jnp.zeros((256, 64), jnp.float32)


def _tc_comb2(accs, dens, xpr, asr, adr, astar, b2r):
    grid = NPAD // 256
    return pl.pallas_call(
        _tc_comb2_body,
        grid=(grid,),
        in_specs=[
            pl.BlockSpec((2, 256, 128), lambda i: (0, i, 0)),
            pl.BlockSpec((2, 1, 256), lambda i: (0, 0, i)),
            pl.BlockSpec((256, 128), lambda i: (i, 0)),
            pl.BlockSpec((1, 256), lambda i: (0, i)),
            pl.BlockSpec((1, 256), lambda i: (0, i)),
            pl.BlockSpec((1, 128), lambda i: (0, 0)),
            pl.BlockSpec((1, 64), lambda i: (0, 0)),
        ],
        out_specs=pl.BlockSpec((256, 128), lambda i: (i, 0)),
        out_shape=jax.ShapeDtypeStruct((NPAD, 128), jnp.float32),
    )(accs, dens, xpr, asr, adr, astar, b2r)


def _tc_mlp_body(u, v, p1u, p1v, pb1, pw2, pb2, out):
    z = jnp.dot(u[...], p1u[...], preferred_element_type=jnp.float32)
    z = z + jnp.dot(v[...], p1v[...], preferred_element_type=jnp.float32)
    z = jnp.maximum(z + pb1[...], 0.0)
    o = jnp.dot(z, pw2[...], preferred_element_type=jnp.float32) + pb2[...]
    out[...] = jax.nn.sigmoid(o)


def _tc_mlp(u, v, P1u, P1v, Pb1r, Pw2p, Pb2r):
    grid = B // 512
    return pl.pallas_call(
        _tc_mlp_body,
        grid=(grid,),
        in_specs=[
            pl.BlockSpec((512, 128), lambda i: (i, 0)),
            pl.BlockSpec((512, 128), lambda i: (i, 0)),
            pl.BlockSpec((128, 128), lambda i: (0, 0)),
            pl.BlockSpec((128, 128), lambda i: (0, 0)),
            pl.BlockSpec((1, 128), lambda i: (0, 0)),
            pl.BlockSpec((128, 8), lambda i: (0, 0)),
            pl.BlockSpec((1, 8), lambda i: (0, 0)),
        ],
        out_specs=pl.BlockSpec((512, 8), lambda i: (i, 0)),
        out_shape=jax.ShapeDtypeStruct((B, 8), jnp.float32),
    )(u, v, P1u, P1v, Pb1r, Pw2p, Pb2r)


def _sc_mesh():
    return plsc.VectorSubcoreMesh(core_axis_name="c", subcore_axis_name="s")


def _sc_weights(edge_f, asT, adT, astar, heads):
    @functools.partial(
        pl.kernel,
        out_type=[
            jax.ShapeDtypeStruct((NBLKT * heads * 128,), jnp.float32),
            jax.ShapeDtypeStruct((2 * heads * NPAD,), jnp.float32),
        ],
        mesh=_sc_mesh(),
        compiler_params=pltpu.CompilerParams(needs_layout_passes=False),
        scratch_types=[
            pltpu.VMEM((NPAD,), jnp.float32),
            pltpu.VMEM((NPAD,), jnp.float32),
            pltpu.VMEM((16,), jnp.float32),
            pltpu.VMEM((2, 256), jnp.int32),
            pltpu.VMEM((2, 128), jnp.int32),
            pltpu.VMEM((2, 128), jnp.float32),
            pltpu.VMEM((3136,), jnp.float32),
            pltpu.VMEM((3136,), jnp.float32),
            pltpu.SemaphoreType.DMA((2,)),
            pltpu.SemaphoreType.DMA((2,)),
            pltpu.SemaphoreType.DMA((2,)),
            pltpu.VMEM_SHARED((heads * NPAD,), jnp.float32),
        ],
    )
    def k(edge_hbm, asT_hbm, adT_hbm, astar_hbm, w_hbm, den_hbm,
          as_t, ad_t, abuf, sdbuf, ibuf, wbuf, zflat, bflat,
          esem, wwsem, dssem, den_sp):
        cid = lax.axis_index("c")
        sid = lax.axis_index("s")
        ebase = (cid * 16 + sid) * EPW

        def _z1(i, _):
            zflat[pl.ds(i * 16, 16)] = jnp.zeros((16,), jnp.float32)
            return 0
        lax.fori_loop(0, 196, _z1, 0)

        def _head(h, _):
            pltpu.sync_copy(asT_hbm.at[pl.ds(h * NPAD, NPAD)], as_t)
            pltpu.sync_copy(adT_hbm.at[pl.ds(h * NPAD, NPAD)], ad_t)
            pltpu.sync_copy(astar_hbm.at[pl.ds(h * 128, 16)], abuf)
            pltpu.sync_copy(zflat, den_sp.at[pl.ds(h * NPAD + sid * STRIPE,
                                                   STRIPE)])
            plsc.subcore_barrier()

            def _start(b, p):
                blkid = (cid * 16 + sid) * NBLK + b
                pltpu.async_copy(edge_hbm.at[pl.ds(blkid * 256, 256)],
                                 sdbuf.at[p], esem.at[p])

            _start(0, 0)

            def _blk(b, _):
                p = lax.rem(b, 2)
                blkid = (cid * 16 + sid) * NBLK + b

                _start(jnp.minimum(b + 1, NBLK - 1), 1 - p)
                pltpu.make_async_copy(edge_hbm.at[pl.ds(0, 256)],
                                      sdbuf.at[p], esem.at[p]).wait()

                def _wv(i, _):
                    pltpu.make_async_copy(
                        wbuf.at[p], w_hbm.at[pl.ds(0, 128)],
                        wwsem.at[p]).wait()
                    pltpu.make_async_copy(
                        wbuf.at[p], den_sp.at[pl.ds(0, 128)],
                        dssem.at[p]).wait()
                    return 0
                lax.fori_loop(0, jnp.where(b >= 2, 1, 0), _wv, 0)

                av16 = abuf[...]
                for g in range(8):
                    sv = sdbuf[p, pl.ds(g * 16, 16)]
                    dv = sdbuf[p, pl.ds(128 + g * 16, 16)]
                    a1 = plsc.load_gather(as_t, [sv])
                    a2 = plsc.load_gather(ad_t, [dv])
                    alpha = _lrelu(a1 + a2)
                    mm = _lrelu(av16 + a2)
                    wbuf[p, pl.ds(g * 16, 16)] = jnp.exp(alpha - mm)
                    ibuf[p, pl.ds(g * 16, 16)] = dv + h * NPAD
                pltpu.async_copy(
                    wbuf.at[p],
                    w_hbm.at[pl.ds((blkid * heads + h) * 128, 128)],
                    wwsem.at[p])
                pltpu.async_copy(wbuf.at[p], den_sp.at[ibuf.at[p]],
                                 dssem.at[p], add=True)
                return 0
            lax.fori_loop(0, NBLK, _blk, 0)
            pltpu.make_async_copy(edge_hbm.at[pl.ds(0, 256)],
                                  sdbuf.at[0], esem.at[0]).wait()
            for p in range(2):
                pltpu.make_async_copy(wbuf.at[p], w_hbm.at[pl.ds(0, 128)],
                                      wwsem.at[p]).wait()
                pltpu.make_async_copy(wbuf.at[p], den_sp.at[pl.ds(0, 128)],
                                      dssem.at[p]).wait()
            plsc.subcore_barrier()
            pltpu.sync_copy(
                den_sp.at[pl.ds(h * NPAD + sid * STRIPE, STRIPE)], bflat)
            pltpu.sync_copy(
                bflat,
                den_hbm.at[pl.ds(cid * heads * NPAD + h * NPAD + sid * STRIPE,
                                 STRIPE)])
            return 0
        lax.fori_loop(0, heads, _head, 0)

    return k(edge_f, asT, adT, astar)


def _sc_aggregate(edge_f, w_f, xp, heads):
    @functools.partial(
        pl.kernel,
        out_type=jax.ShapeDtypeStruct((2, NRANGE, NR, 128), jnp.float32),
        mesh=_sc_mesh(),
        compiler_params=pltpu.CompilerParams(needs_layout_passes=False),
        scratch_types=[
            pltpu.VMEM((2, 256), jnp.int32),
            pltpu.VMEM((2, 4 * 128), jnp.float32),
            pltpu.VMEM((256,), jnp.int32),
            pltpu.VMEM((256,), jnp.int32),
            pltpu.VMEM((4 * 256,), jnp.float32),
            pltpu.VMEM((128,), jnp.int32),
            pltpu.VMEM((128, 128), jnp.float32),
            pltpu.SemaphoreType.DMA((2,)),
            pltpu.SemaphoreType.DMA((2,)),
            pltpu.VMEM_SHARED((NR, 128), jnp.float32),
        ],
    )
    def k(edge_hbm, w_hbm, xp_hbm, acc_hbm,
          sdbuf, wstg, csrc, cdst, cw, fdst, rows, esem, wsem, acc_sp):
        cid = lax.axis_index("c")
        sid = lax.axis_index("s")
        nh = heads

        for g in range(16):
            csrc[pl.ds(g * 16, 16)] = jnp.zeros((16,), jnp.int32)
            cdst[pl.ds(g * 16, 16)] = jnp.zeros((16,), jnp.int32)
        for g in range(16 * nh):
            cw[pl.ds(g * 16, 16)] = jnp.zeros((16,), jnp.float32)

        def _flush():
            for j in range(8):
                fdst[pl.ds(j * 16, 16)] = cdst[pl.ds(j * 16, 16)]
            pltpu.sync_copy(xp_hbm.at[csrc.at[pl.ds(0, 128)]], rows)

            def _scale(g, _):
                wvecs = [cw[pl.ds(q * 256 + g * 16, 16)] for q in range(nh)]
                for e in range(16):
                    i = g * 16 + e
                    for q in range(nh):
                        wv = jnp.full((16,), wvecs[q][e], jnp.float32)
                        rng = (range(2 * q, 2 * q + 2) if nh == 4
                               else range(8))
                        for j in rng:
                            rows[i, pl.ds(j * 16, 16)] = (
                                rows[i, pl.ds(j * 16, 16)] * wv)
                return 0
            lax.fori_loop(0, 8, _scale, 0)
            pltpu.sync_copy(rows, acc_sp.at[fdst], add=True)
            for j in range(8):
                csrc[pl.ds(j * 16, 16)] = csrc[pl.ds(128 + j * 16, 16)]
                cdst[pl.ds(j * 16, 16)] = cdst[pl.ds(128 + j * 16, 16)]
            for q in range(nh):
                for j in range(8):
                    cw[pl.ds(q * 256 + j * 16, 16)] = (
                        cw[pl.ds(q * 256 + 128 + j * 16, 16)])

        def _range(r, _):
            lo = r * NR

            def _z0(i, _):
                for j in range(8):
                    rows[i, pl.ds(j * 16, 16)] = jnp.zeros((16,), jnp.float32)
                return 0
            lax.fori_loop(0, 112, _z0, 0)

            def _zacc(i, _):
                pltpu.sync_copy(rows.at[pl.ds(0, 112)],
                                acc_sp.at[pl.ds(sid * RSTRIPE + i * 112, 112)])
                return 0
            lax.fori_loop(0, 7, _zacc, 0)
            plsc.subcore_barrier()

            def _start(b, p):
                blkid = (cid * 16 + sid) * NBLK + b
                pltpu.async_copy(edge_hbm.at[pl.ds(blkid * 256, 256)],
                                 sdbuf.at[p], esem.at[p])
                pltpu.async_copy(
                    w_hbm.at[pl.ds(blkid * nh * 128, nh * 128)],
                    wstg.at[p, pl.ds(0, nh * 128)], wsem.at[p])

            _start(0, 0)

            def _blk(b, off):
                p = lax.rem(b, 2)
                _start(jnp.minimum(b + 1, NBLK - 1), 1 - p)
                pltpu.make_async_copy(edge_hbm.at[pl.ds(0, 256)],
                                      sdbuf.at[p], esem.at[p]).wait()
                pltpu.make_async_copy(
                    w_hbm.at[pl.ds(0, nh * 128)],
                    wstg.at[p, pl.ds(0, nh * 128)], wsem.at[p]).wait()
                for g in range(8):
                    sv = sdbuf[p, pl.ds(g * 16, 16)]
                    dv = sdbuf[p, pl.ds(128 + g * 16, 16)]
                    msk = (dv >= lo) & (dv < lo + NR)
                    plsc.store_compressed(csrc.at[pl.ds(off, 16)], sv,
                                          mask=msk)
                    plsc.store_compressed(cdst.at[pl.ds(off, 16)], dv - lo,
                                          mask=msk)
                    for q in range(nh):
                        wv = wstg[p, pl.ds(q * 128 + g * 16, 16)]
                        plsc.store_compressed(
                            cw.at[pl.ds(q * 256 + off, 16)], wv, mask=msk)
                    cnt = plsc.all_reduce_population_count(msk)
                    off = off + cnt[0]

                def _doflush(i, _):
                    _flush()
                    return 0
                nfl = jnp.where(off >= 128, 1, 0)
                lax.fori_loop(0, nfl, _doflush, 0)
                off = jnp.where(off >= 128, off - 128, off)
                return off
            off = lax.fori_loop(0, NBLK, _blk, jnp.int32(0))
            pltpu.make_async_copy(edge_hbm.at[pl.ds(0, 256)],
                                  sdbuf.at[0], esem.at[0]).wait()
            pltpu.make_async_copy(w_hbm.at[pl.ds(0, nh * 128)],
                                  wstg.at[0, pl.ds(0, nh * 128)],
                                  wsem.at[0]).wait()

            iota = lax.iota(jnp.int32, 16)
            for g in range(8):
                keep = (iota + g * 16) < off
                for q in range(nh):
                    wv = cw[pl.ds(q * 256 + g * 16, 16)]
                    cw[pl.ds(q * 256 + g * 16, 16)] = jnp.where(
                        keep, wv, jnp.zeros((16,), jnp.float32))
            _flush()
            plsc.subcore_barrier()

            def _dump(i, _):
                o = sid * RSTRIPE + i * 112
                pltpu.sync_copy(acc_sp.at[pl.ds(o, 112)],
                                rows.at[pl.ds(0, 112)])
                pltpu.sync_copy(rows.at[pl.ds(0, 112)],
                                acc_hbm.at[cid, r, pl.ds(o, 112)])
                return 0
            lax.fori_loop(0, 7, _dump, 0)
            return 0
        lax.fori_loop(0, NRANGE, _range, 0)

    return k(edge_f, w_f, xp)


def _sc_gather(h2, uid, vid):
    @functools.partial(
        pl.kernel,
        out_type=[
            jax.ShapeDtypeStruct((B, 128), jnp.float32),
            jax.ShapeDtypeStruct((B, 128), jnp.float32),
        ],
        mesh=_sc_mesh(),
        compiler_params=pltpu.CompilerParams(needs_layout_passes=False),
        scratch_types=[
            pltpu.VMEM((128,), jnp.int32),
            pltpu.VMEM((128, 128), jnp.float32),
        ],
    )
    def k(h2_hbm, uid_hbm, vid_hbm, u_hbm, v_hbm, ibuf, rbuf):
        cid = lax.axis_index("c")
        sid = lax.axis_index("s")
        base = (cid * 16 + sid) * (B // NWORK)

        def _blk(b, _):
            off = base + b * 128
            pltpu.sync_copy(uid_hbm.at[pl.ds(off, 128)], ibuf)
            pltpu.sync_copy(h2_hbm.at[ibuf], rbuf)
            pltpu.sync_copy(rbuf, u_hbm.at[pl.ds(off, 128)])
            pltpu.sync_copy(vid_hbm.at[pl.ds(off, 128)], ibuf)
            pltpu.sync_copy(h2_hbm.at[ibuf], rbuf)
            pltpu.sync_copy(rbuf, v_hbm.at[pl.ds(off, 128)])
            return 0
        lax.fori_loop(0, (B // NWORK) // 128, _blk, 0)

    return k(h2, uid, vid)


def kernel(x, edge_index, user_indices, item_indices, table, W1, a_src1,
           a_dst1, b1, W2, a_src2, a_dst2, b2, Pw1, Pb1, Pw2, Pb2):
    f32 = jnp.float32
    table_p = jnp.pad(table, ((0, NPAD - N), (0, 0)))
    npad_ids = jnp.arange(EPAD - E, dtype=jnp.int32)
    pad_src = npad_ids % N
    pad_dst = N + (npad_ids % (NPAD - N))
    edge_p = jnp.concatenate(
        [edge_index, jnp.stack([pad_src, pad_dst])], axis=1)
    edge_f = edge_p.reshape(2, NBLKT, 128).transpose(1, 0, 2).reshape(-1)

    eye4 = jnp.eye(4, dtype=f32)
    As1T = (eye4[:, :, None] * a_src1[None]).reshape(4, 128)
    Ad1T = (eye4[:, :, None] * a_dst1[None]).reshape(4, 128)
    b1r = b1.reshape(1, 128)
    b2r = b2.reshape(1, 64)
    P1u = jnp.pad(Pw1[:64], ((0, 64), (0, 0)))
    P1v = jnp.pad(Pw1[64:], ((0, 64), (0, 0)))
    Pb1r = Pb1.reshape(1, 128)
    Pw2p = jnp.pad(Pw2, ((0, 0), (0, 7)))
    Pb2r = jnp.pad(Pb2.reshape(1, 1), ((0, 0), (0, 7)))

    xp1, asT1, adT1, astar1 = _tc_pre1(table_p, W1, As1T, Ad1T)
    w1f, den1 = _sc_weights(edge_f, asT1.reshape(-1), adT1.reshape(-1),
                            astar1.reshape(-1), 4)
    acc1 = _sc_aggregate(edge_f, w1f, xp1, 4)
    xp2, asT2, adT2, astar2 = _tc_comb1(
        acc1.reshape(2, NPAD, 128), den1.reshape(2, 4, NPAD), xp1,
        asT1, adT1, astar1, b1r, W2, a_src2, a_dst2)
    w2f, den2 = _sc_weights(edge_f, asT2.reshape(-1), adT2.reshape(-1),
                            astar2.reshape(-1), 1)
    acc2 = _sc_aggregate(edge_f, w2f, xp2, 1)
    h2 = _tc_comb2(acc2.reshape(2, NPAD, 128), den2.reshape(2, 1, NPAD),
                   xp2, asT2, adT2, astar2, b2r)
    u, v = _sc_gather(h2, user_indices, item_indices)
    out = _tc_mlp(u, v, P1u, P1v, Pb1r, Pw2p, Pb2r)
    return out[:, 0]


# speedup vs baseline: 1.3247x; 1.0520x over previous
import functools

import jax
import jax.numpy as jnp
from jax import lax
from jax.experimental import pallas as pl
from jax.experimental.pallas import tpu as pltpu
from jax.experimental.pallas import tpu_sc as plsc

N = 50000
E = 800000
B = 16384
D = 64

NPAD = 50176
EPAD = 802816
NWORK = 32
EPW = EPAD // NWORK
NBLK = EPW // 128
NBLKT = NWORK * NBLK
STRIPE = NPAD // 16
NRANGE = 4
NR = NPAD // NRANGE
RSTRIPE = NR // 16


def _lrelu(x):
    return jnp.maximum(x, 0.2 * x)


def _tc_pre1_body(tbl, w1, ast, adt, xpr, asr, adr, astar):
    xp = jnp.dot(tbl[...], w1[...], preferred_element_type=jnp.float32)
    xpr[...] = xp
    a_s = lax.dot_general(ast[...], xp, (((1,), (1,)), ((), ())),
                          preferred_element_type=jnp.float32)
    a_d = lax.dot_general(adt[...], xp, (((1,), (1,)), ((), ())),
                          preferred_element_type=jnp.float32)
    asr[...] = a_s
    adr[...] = a_d
    bm = jnp.max(a_s, axis=1)
    prev = jnp.where(pl.program_id(0) == 0,
                     jnp.full((4, 128), -3.4e38, jnp.float32), astar[...])
    astar[...] = jnp.maximum(prev, bm[:, None])


def _tc_pre1(table_p, W1, As1T, Ad1T):
    grid = NPAD // 256
    return pl.pallas_call(
        _tc_pre1_body,
        grid=(grid,),
        in_specs=[
            pl.BlockSpec((256, D), lambda i: (i, 0)),
            pl.BlockSpec((D, 128), lambda i: (0, 0)),
            pl.BlockSpec((4, 128), lambda i: (0, 0)),
            pl.BlockSpec((4, 128), lambda i: (0, 0)),
        ],
        out_specs=[
            pl.BlockSpec((256, 128), lambda i: (i, 0)),
            pl.BlockSpec((4, 256), lambda i: (0, i)),
            pl.BlockSpec((4, 256), lambda i: (0, i)),
            pl.BlockSpec((4, 128), lambda i: (0, 0)),
        ],
        out_shape=[
            jax.ShapeDtypeStruct((NPAD, 128), jnp.float32),
            jax.ShapeDtypeStruct((4, NPAD), jnp.float32),
            jax.ShapeDtypeStruct((4, NPAD), jnp.float32),
            jax.ShapeDtypeStruct((4, 128), jnp.float32),
        ],
    )(table_p, W1, As1T, Ad1T)


def _tc_comb1_body(accs, dens, xpr, asr, adr, astar, b1r, w2, as2t, ad2t,
                   xpr2, asr2, adr2, astar2):
    num = accs[0] + accs[1]
    den = dens[0] + dens[1]
    a_s = asr[...]
    a_d = adr[...]
    m = _lrelu(astar[:, 0:1] + a_d)
    ws = jnp.exp(_lrelu(a_s + a_d) - m)
    xp = xpr[...]
    cols = []
    for h in range(4):
        sl = slice(h * 32, (h + 1) * 32)
        numh = num[:, sl] + ws[h][:, None] * xp[:, sl]
        cols.append(numh / (den[h] + ws[h] + 1e-16)[:, None])
    h1 = jnp.concatenate(cols, axis=1) + b1r[...]
    h1 = jnp.where(h1 > 0, h1, jnp.exp(jnp.minimum(h1, 0.0)) - 1.0)
    xp2 = jnp.dot(h1, w2[...], preferred_element_type=jnp.float32)
    xpr2[:, 0:64] = xp2
    xpr2[:, 64:128] = jnp.zeros((256, 64), jnp.float32)
    a_s2 = lax.dot_general(as2t[...], xp2, (((1,), (1,)), ((), ())),
                           preferred_element_type=jnp.float32)
    a_d2 = lax.dot_general(ad2t[...], xp2, (((1,), (1,)), ((), ())),
                           preferred_element_type=jnp.float32)
    asr2[...] = a_s2
    adr2[...] = a_d2
    bm = jnp.max(a_s2, axis=1)
    prev = jnp.where(pl.program_id(0) == 0,
                     jnp.full((1, 128), -3.4e38, jnp.float32), astar2[...])
    astar2[...] = jnp.maximum(prev, bm[:, None])


def _tc_comb1(accs, dens, xpr, asr, adr, astar, b1r, W2, As2T, Ad2T):
    grid = NPAD // 256
    return pl.pallas_call(
        _tc_comb1_body,
        grid=(grid,),
        in_specs=[
            pl.BlockSpec((2, 256, 128), lambda i: (0, i, 0)),
            pl.BlockSpec((2, 4, 256), lambda i: (0, 0, i)),
            pl.BlockSpec((256, 128), lambda i: (i, 0)),
            pl.BlockSpec((4, 256), lambda i: (0, i)),
            pl.BlockSpec((4, 256), lambda i: (0, i)),
            pl.BlockSpec((4, 128), lambda i: (0, 0)),
            pl.BlockSpec((1, 128), lambda i: (0, 0)),
            pl.BlockSpec((128, 64), lambda i: (0, 0)),
            pl.BlockSpec((1, 64), lambda i: (0, 0)),
            pl.BlockSpec((1, 64), lambda i: (0, 0)),
        ],
        out_specs=[
            pl.BlockSpec((256, 128), lambda i: (i, 0)),
            pl.BlockSpec((1, 256), lambda i: (0, i)),
            pl.BlockSpec((1, 256), lambda i: (0, i)),
            pl.BlockSpec((1, 128), lambda i: (0, 0)),
        ],
        out_shape=[
            jax.ShapeDtypeStruct((NPAD, 128), jnp.float32),
            jax.ShapeDtypeStruct((1, NPAD), jnp.float32),
            jax.ShapeDtypeStruct((1, NPAD), jnp.float32),
            jax.ShapeDtypeStruct((1, 128), jnp.float32),
        ],
    )(accs, dens, xpr, asr, adr, astar, b1r, W2, As2T, Ad2T)


def _tc_comb2_body(accs, dens, xpr, asr, adr, astar, b2r, h2):
    num = accs[0] + accs[1]
    den = dens[0] + dens[1]
    a_s = asr[...]
    a_d = adr[...]
    m = _lrelu(astar[:, 0:1] + a_d)
    ws = jnp.exp(_lrelu(a_s + a_d) - m)
    xp = xpr[...]
    out = (num[:, 0:64] + ws[0][:, None] * xp[:, 0:64])
    out = out / (den[0] + ws[0] + 1e-16)[:, None] + b2r[...]
    h2[:, 0:64] = out
    h2[:, 64:128] = jnp.zeros((256, 64), jnp.float32)


def _tc_comb2(accs, dens, xpr, asr, adr, astar, b2r):
    grid = NPAD // 256
    return pl.pallas_call(
        _tc_comb2_body,
        grid=(grid,),
        in_specs=[
            pl.BlockSpec((2, 256, 128), lambda i: (0, i, 0)),
            pl.BlockSpec((2, 1, 256), lambda i: (0, 0, i)),
            pl.BlockSpec((256, 128), lambda i: (i, 0)),
            pl.BlockSpec((1, 256), lambda i: (0, i)),
            pl.BlockSpec((1, 256), lambda i: (0, i)),
            pl.BlockSpec((1, 128), lambda i: (0, 0)),
            pl.BlockSpec((1, 64), lambda i: (0, 0)),
        ],
        out_specs=pl.BlockSpec((256, 128), lambda i: (i, 0)),
        out_shape=jax.ShapeDtypeStruct((NPAD, 128), jnp.float32),
    )(accs, dens, xpr, asr, adr, astar, b2r)


def _tc_mlp_body(u, v, p1u, p1v, pb1, pw2, pb2, out):
    z = jnp.dot(u[...], p1u[...], preferred_element_type=jnp.float32)
    z = z + jnp.dot(v[...], p1v[...], preferred_element_type=jnp.float32)
    z = jnp.maximum(z + pb1[...], 0.0)
    o = jnp.dot(z, pw2[...], preferred_element_type=jnp.float32) + pb2[...]
    out[...] = jax.nn.sigmoid(o)


def _tc_mlp(u, v, P1u, P1v, Pb1r, Pw2p, Pb2r):
    grid = B // 512
    return pl.pallas_call(
        _tc_mlp_body,
        grid=(grid,),
        in_specs=[
            pl.BlockSpec((512, 128), lambda i: (i, 0)),
            pl.BlockSpec((512, 128), lambda i: (i, 0)),
            pl.BlockSpec((128, 128), lambda i: (0, 0)),
            pl.BlockSpec((128, 128), lambda i: (0, 0)),
            pl.BlockSpec((1, 128), lambda i: (0, 0)),
            pl.BlockSpec((128, 8), lambda i: (0, 0)),
            pl.BlockSpec((1, 8), lambda i: (0, 0)),
        ],
        out_specs=pl.BlockSpec((512, 8), lambda i: (i, 0)),
        out_shape=jax.ShapeDtypeStruct((B, 8), jnp.float32),
    )(u, v, P1u, P1v, Pb1r, Pw2p, Pb2r)


def _sc_mesh():
    return plsc.VectorSubcoreMesh(core_axis_name="c", subcore_axis_name="s")


def _sc_weights(edge_f, asT, adT, astar, heads):
    @functools.partial(
        pl.kernel,
        out_type=[
            jax.ShapeDtypeStruct((NBLKT * heads * 128,), jnp.float32),
            jax.ShapeDtypeStruct((2 * heads * NPAD,), jnp.float32),
        ],
        mesh=_sc_mesh(),
        compiler_params=pltpu.CompilerParams(needs_layout_passes=False),
        scratch_types=[
            pltpu.VMEM((NPAD,), jnp.float32),
            pltpu.VMEM((NPAD,), jnp.float32),
            pltpu.VMEM((16,), jnp.float32),
            pltpu.VMEM((2, 256), jnp.int32),
            pltpu.VMEM((2, 128), jnp.int32),
            pltpu.VMEM((2, 128), jnp.float32),
            pltpu.VMEM((3136,), jnp.float32),
            pltpu.VMEM((3136,), jnp.float32),
            pltpu.SemaphoreType.DMA((2,)),
            pltpu.SemaphoreType.DMA((2,)),
            pltpu.SemaphoreType.DMA((2,)),
            pltpu.VMEM_SHARED((heads * NPAD,), jnp.float32),
        ],
    )
    def k(edge_hbm, asT_hbm, adT_hbm, astar_hbm, w_hbm, den_hbm,
          as_t, ad_t, abuf, sdbuf, ibuf, wbuf, zflat, bflat,
          esem, wwsem, dssem, den_sp):
        cid = lax.axis_index("c")
        sid = lax.axis_index("s")
        ebase = (cid * 16 + sid) * EPW

        def _z1(i, _):
            zflat[pl.ds(i * 16, 16)] = jnp.zeros((16,), jnp.float32)
            return 0
        lax.fori_loop(0, 196, _z1, 0)

        def _head(h, _):
            pltpu.sync_copy(asT_hbm.at[pl.ds(h * NPAD, NPAD)], as_t)
            pltpu.sync_copy(adT_hbm.at[pl.ds(h * NPAD, NPAD)], ad_t)
            pltpu.sync_copy(astar_hbm.at[pl.ds(h * 128, 16)], abuf)
            pltpu.sync_copy(zflat, den_sp.at[pl.ds(h * NPAD + sid * STRIPE,
                                                   STRIPE)])
            plsc.subcore_barrier()

            def _start(b, p):
                blkid = (cid * 16 + sid) * NBLK + b
                pltpu.async_copy(edge_hbm.at[pl.ds(blkid * 256, 256)],
                                 sdbuf.at[p], esem.at[p])

            _start(0, 0)

            def _blk(b, _):
                p = lax.rem(b, 2)
                blkid = (cid * 16 + sid) * NBLK + b

                _start(jnp.minimum(b + 1, NBLK - 1), 1 - p)
                pltpu.make_async_copy(edge_hbm.at[pl.ds(0, 256)],
                                      sdbuf.at[p], esem.at[p]).wait()

                def _wv(i, _):
                    pltpu.make_async_copy(
                        wbuf.at[p], w_hbm.at[pl.ds(0, 128)],
                        wwsem.at[p]).wait()
                    pltpu.make_async_copy(
                        wbuf.at[p], den_sp.at[pl.ds(0, 128)],
                        dssem.at[p]).wait()
                    return 0
                lax.fori_loop(0, jnp.where(b >= 2, 1, 0), _wv, 0)

                av16 = abuf[...]
                for g in range(8):
                    sv = sdbuf[p, pl.ds(g * 16, 16)]
                    dv = sdbuf[p, pl.ds(128 + g * 16, 16)]
                    a1 = plsc.load_gather(as_t, [sv])
                    a2 = plsc.load_gather(ad_t, [dv])
                    alpha = _lrelu(a1 + a2)
                    mm = _lrelu(av16 + a2)
                    wbuf[p, pl.ds(g * 16, 16)] = jnp.exp(alpha - mm)
                    ibuf[p, pl.ds(g * 16, 16)] = dv + h * NPAD
                pltpu.async_copy(
                    wbuf.at[p],
                    w_hbm.at[pl.ds((blkid * heads + h) * 128, 128)],
                    wwsem.at[p])
                pltpu.async_copy(wbuf.at[p], den_sp.at[ibuf.at[p]],
                                 dssem.at[p], add=True)
                return 0
            lax.fori_loop(0, NBLK, _blk, 0)
            pltpu.make_async_copy(edge_hbm.at[pl.ds(0, 256)],
                                  sdbuf.at[0], esem.at[0]).wait()
            for p in range(2):
                pltpu.make_async_copy(wbuf.at[p], w_hbm.at[pl.ds(0, 128)],
                                      wwsem.at[p]).wait()
                pltpu.make_async_copy(wbuf.at[p], den_sp.at[pl.ds(0, 128)],
                                      dssem.at[p]).wait()
            plsc.subcore_barrier()
            pltpu.sync_copy(
                den_sp.at[pl.ds(h * NPAD + sid * STRIPE, STRIPE)], bflat)
            pltpu.sync_copy(
                bflat,
                den_hbm.at[pl.ds(cid * heads * NPAD + h * NPAD + sid * STRIPE,
                                 STRIPE)])
            return 0
        lax.fori_loop(0, heads, _head, 0)

    return k(edge_f, asT, adT, astar)


def _sc_aggregate(edge_f, w_f, xp, heads):
    @functools.partial(
        pl.kernel,
        out_type=jax.ShapeDtypeStruct((2, NRANGE, NR, 128), jnp.float32),
        mesh=_sc_mesh(),
        compiler_params=pltpu.CompilerParams(needs_layout_passes=False),
        scratch_types=[
            pltpu.VMEM((2, 512), jnp.int32),
            pltpu.VMEM((2, 4 * 256), jnp.float32),
            pltpu.VMEM((256,), jnp.int32),
            pltpu.VMEM((256,), jnp.int32),
            pltpu.VMEM((4 * 256,), jnp.float32),
            pltpu.VMEM((128,), jnp.int32),
            pltpu.VMEM((128, 128), jnp.float32),
            pltpu.SemaphoreType.DMA((2,)),
            pltpu.SemaphoreType.DMA((2,)),
            pltpu.VMEM_SHARED((NR, 128), jnp.float32),
        ],
    )
    def k(edge_hbm, w_hbm, xp_hbm, acc_hbm,
          sdbuf, wstg, csrc, cdst, cw, fdst, rows, esem, wsem, acc_sp):
        cid = lax.axis_index("c")
        sid = lax.axis_index("s")
        nh = heads

        for g in range(16):
            csrc[pl.ds(g * 16, 16)] = jnp.zeros((16,), jnp.int32)
            cdst[pl.ds(g * 16, 16)] = jnp.zeros((16,), jnp.int32)
        for g in range(16 * nh):
            cw[pl.ds(g * 16, 16)] = jnp.zeros((16,), jnp.float32)

        def _flush():
            for j in range(8):
                fdst[pl.ds(j * 16, 16)] = cdst[pl.ds(j * 16, 16)]
            pltpu.sync_copy(xp_hbm.at[csrc.at[pl.ds(0, 128)]], rows)

            def _scale(g, _):
                wvecs = [cw[pl.ds(q * 256 + g * 16, 16)] for q in range(nh)]
                for e in range(16):
                    i = g * 16 + e
                    for q in range(nh):
                        wv = jnp.full((16,), wvecs[q][e], jnp.float32)
                        rng = (range(2 * q, 2 * q + 2) if nh == 4
                               else range(8))
                        for j in rng:
                            rows[i, pl.ds(j * 16, 16)] = (
                                rows[i, pl.ds(j * 16, 16)] * wv)
                return 0
            lax.fori_loop(0, 8, _scale, 0)
            pltpu.sync_copy(rows, acc_sp.at[fdst], add=True)
            for j in range(8):
                csrc[pl.ds(j * 16, 16)] = csrc[pl.ds(128 + j * 16, 16)]
                cdst[pl.ds(j * 16, 16)] = cdst[pl.ds(128 + j * 16, 16)]
            for q in range(nh):
                for j in range(8):
                    cw[pl.ds(q * 256 + j * 16, 16)] = (
                        cw[pl.ds(q * 256 + 128 + j * 16, 16)])

        def _range(r, _):
            lo = r * NR

            def _z0(i, _):
                for j in range(8):
                    rows[i, pl.ds(j * 16, 16)] = jnp.zeros((16,), jnp.float32)
                return 0
            lax.fori_loop(0, 112, _z0, 0)

            def _zacc(i, _):
                pltpu.sync_copy(rows.at[pl.ds(0, 112)],
                                acc_sp.at[pl.ds(sid * RSTRIPE + i * 112, 112)])
                return 0
            lax.fori_loop(0, 7, _zacc, 0)
            plsc.subcore_barrier()

            def _start(sb, p):
                blkid = (cid * 16 + sid) * NBLK + sb * 2
                pltpu.async_copy(edge_hbm.at[pl.ds(blkid * 256, 512)],
                                 sdbuf.at[p], esem.at[p])
                pltpu.async_copy(
                    w_hbm.at[pl.ds(blkid * nh * 128, nh * 256)],
                    wstg.at[p, pl.ds(0, nh * 256)], wsem.at[p])

            _start(0, 0)
            NSB = NBLK // 2

            def _blk(sb, off):
                p = lax.rem(sb, 2)
                _start(jnp.minimum(sb + 1, NSB - 1), 1 - p)
                pltpu.make_async_copy(edge_hbm.at[pl.ds(0, 512)],
                                      sdbuf.at[p], esem.at[p]).wait()
                pltpu.make_async_copy(
                    w_hbm.at[pl.ds(0, nh * 256)],
                    wstg.at[p, pl.ds(0, nh * 256)], wsem.at[p]).wait()
                for half in range(2):
                    for g in range(8):
                        sv = sdbuf[p, pl.ds(half * 256 + g * 16, 16)]
                        dv = sdbuf[p, pl.ds(half * 256 + 128 + g * 16, 16)]
                        msk = (dv >= lo) & (dv < lo + NR)
                        plsc.store_compressed(csrc.at[pl.ds(off, 16)], sv,
                                              mask=msk)
                        plsc.store_compressed(cdst.at[pl.ds(off, 16)],
                                              dv - lo, mask=msk)
                        for q in range(nh):
                            wv = wstg[p, pl.ds(half * nh * 128 + q * 128
                                               + g * 16, 16)]
                            plsc.store_compressed(
                                cw.at[pl.ds(q * 256 + off, 16)], wv,
                                mask=msk)
                        cnt = plsc.all_reduce_population_count(msk)
                        off = off + cnt[0]

                    def _doflush(i, _):
                        _flush()
                        return 0
                    nfl = jnp.where(off >= 128, 1, 0)
                    lax.fori_loop(0, nfl, _doflush, 0)
                    off = jnp.where(off >= 128, off - 128, off)
                return off
            off = lax.fori_loop(0, NBLK // 2, _blk, jnp.int32(0))
            pltpu.make_async_copy(edge_hbm.at[pl.ds(0, 512)],
                                  sdbuf.at[0], esem.at[0]).wait()
            pltpu.make_async_copy(w_hbm.at[pl.ds(0, nh * 256)],
                                  wstg.at[0, pl.ds(0, nh * 256)],
                                  wsem.at[0]).wait()

            iota = lax.iota(jnp.int32, 16)
            for g in range(8):
                keep = (iota + g * 16) < off
                for q in range(nh):
                    wv = cw[pl.ds(q * 256 + g * 16, 16)]
                    cw[pl.ds(q * 256 + g * 16, 16)] = jnp.where(
                        keep, wv, jnp.zeros((16,), jnp.float32))
            _flush()
            plsc.subcore_barrier()

            def _dump(i, _):
                o = sid * RSTRIPE + i * 112
                pltpu.sync_copy(acc_sp.at[pl.ds(o, 112)],
                                rows.at[pl.ds(0, 112)])
                pltpu.sync_copy(rows.at[pl.ds(0, 112)],
                                acc_hbm.at[cid, r, pl.ds(o, 112)])
                return 0
            lax.fori_loop(0, 7, _dump, 0)
            return 0
        lax.fori_loop(0, NRANGE, _range, 0)

    return k(edge_f, w_f, xp)


def _sc_gather(h2, uid, vid):
    @functools.partial(
        pl.kernel,
        out_type=[
            jax.ShapeDtypeStruct((B, 128), jnp.float32),
            jax.ShapeDtypeStruct((B, 128), jnp.float32),
        ],
        mesh=_sc_mesh(),
        compiler_params=pltpu.CompilerParams(needs_layout_passes=False),
        scratch_types=[
            pltpu.VMEM((128,), jnp.int32),
            pltpu.VMEM((128, 128), jnp.float32),
        ],
    )
    def k(h2_hbm, uid_hbm, vid_hbm, u_hbm, v_hbm, ibuf, rbuf):
        cid = lax.axis_index("c")
        sid = lax.axis_index("s")
        base = (cid * 16 + sid) * (B // NWORK)

        def _blk(b, _):
            off = base + b * 128
            pltpu.sync_copy(uid_hbm.at[pl.ds(off, 128)], ibuf)
            pltpu.sync_copy(h2_hbm.at[ibuf], rbuf)
            pltpu.sync_copy(rbuf, u_hbm.at[pl.ds(off, 128)])
            pltpu.sync_copy(vid_hbm.at[pl.ds(off, 128)], ibuf)
            pltpu.sync_copy(h2_hbm.at[ibuf], rbuf)
            pltpu.sync_copy(rbuf, v_hbm.at[pl.ds(off, 128)])
            return 0
        lax.fori_loop(0, (B // NWORK) // 128, _blk, 0)

    return k(h2, uid, vid)


def kernel(x, edge_index, user_indices, item_indices, table, W1, a_src1,
           a_dst1, b1, W2, a_src2, a_dst2, b2, Pw1, Pb1, Pw2, Pb2):
    f32 = jnp.float32
    table_p = jnp.pad(table, ((0, NPAD - N), (0, 0)))
    npad_ids = jnp.arange(EPAD - E, dtype=jnp.int32)
    pad_src = npad_ids % N
    pad_dst = N + (npad_ids % (NPAD - N))
    edge_p = jnp.concatenate(
        [edge_index, jnp.stack([pad_src, pad_dst])], axis=1)
    edge_f = edge_p.reshape(2, NBLKT, 128).transpose(1, 0, 2).reshape(-1)

    eye4 = jnp.eye(4, dtype=f32)
    As1T = (eye4[:, :, None] * a_src1[None]).reshape(4, 128)
    Ad1T = (eye4[:, :, None] * a_dst1[None]).reshape(4, 128)
    b1r = b1.reshape(1, 128)
    b2r = b2.reshape(1, 64)
    P1u = jnp.pad(Pw1[:64], ((0, 64), (0, 0)))
    P1v = jnp.pad(Pw1[64:], ((0, 64), (0, 0)))
    Pb1r = Pb1.reshape(1, 128)
    Pw2p = jnp.pad(Pw2, ((0, 0), (0, 7)))
    Pb2r = jnp.pad(Pb2.reshape(1, 1), ((0, 0), (0, 7)))

    xp1, asT1, adT1, astar1 = _tc_pre1(table_p, W1, As1T, Ad1T)
    w1f, den1 = _sc_weights(edge_f, asT1.reshape(-1), adT1.reshape(-1),
                            astar1.reshape(-1), 4)
    acc1 = _sc_aggregate(edge_f, w1f, xp1, 4)
    xp2, asT2, adT2, astar2 = _tc_comb1(
        acc1.reshape(2, NPAD, 128), den1.reshape(2, 4, NPAD), xp1,
        asT1, adT1, astar1, b1r, W2, a_src2, a_dst2)
    w2f, den2 = _sc_weights(edge_f, asT2.reshape(-1), adT2.reshape(-1),
                            astar2.reshape(-1), 1)
    acc2 = _sc_aggregate(edge_f, w2f, xp2, 1)
    h2 = _tc_comb2(acc2.reshape(2, NPAD, 128), den2.reshape(2, 1, NPAD),
                   xp2, asT2, adT2, astar2, b2r)
    u, v = _sc_gather(h2, user_indices, item_indices)
    out = _tc_mlp(u, v, P1u, P1v, Pb1r, Pw2p, Pb2r)
    return out[:, 0]
